# baseline (speedup 1.0000x reference)
import functools

import jax
import jax.numpy as jnp
from jax import lax
from jax.experimental import pallas as pl
from jax.experimental.pallas import tpu as pltpu
from jax.experimental.pallas import tpu_sc as plsc


def _make_sc_scatter(S, E):
    info = plsc.get_sparse_core_info()
    NC, NS = 1, info.num_subcores
    NW = NC * NS
    epw = E // NW
    zpw = (S * S) // NS
    mesh = plsc.VectorSubcoreMesh(core_axis_name="c", subcore_axis_name="s",
                                  num_cores=NC)

    @functools.partial(
        pl.kernel, mesh=mesh,
        out_type=jax.ShapeDtypeStruct((NC * S * S,), jnp.float32),
        scratch_types=[
            pltpu.VMEM((epw,), jnp.int32),
            pltpu.VMEM((epw,), jnp.int32),
            pltpu.VMEM((epw,), jnp.int32),
            pltpu.VMEM((epw,), jnp.float32),
            pltpu.VMEM_SHARED((S * S,), jnp.float32),
            pltpu.SemaphoreType.DMA,
        ],
    )
    def sc_scatter(src_hbm, dst_hbm, ones_hbm, zeros_hbm, out_hbm,
                   dst_v, src_v, fidx_v, ones_v, acc, sem):
        cid = lax.axis_index("c")
        sid = lax.axis_index("s")
        wid = sid * NC + cid
        base = wid * epw
        c0 = pltpu.async_copy(zeros_hbm.at[pl.ds(sid * zpw, zpw)],
                              acc.at[pl.ds(sid * zpw, zpw)], sem)
        c1 = pltpu.async_copy(dst_hbm.at[pl.ds(base, epw)], dst_v, sem)
        c2 = pltpu.async_copy(src_hbm.at[pl.ds(base, epw)], src_v, sem)
        c3 = pltpu.async_copy(ones_hbm.at[pl.ds(base, epw)], ones_v, sem)
        c0.wait()
        c1.wait()
        c2.wait()
        c3.wait()
        for j in range(epw // 16):
            sl = pl.ds(j * 16, 16)
            fidx_v[sl] = dst_v[sl] * S + src_v[sl]
        plsc.subcore_barrier()
        pltpu.sync_copy(ones_v, acc.at[fidx_v], add=True)
        plsc.subcore_barrier()
        pltpu.sync_copy(acc.at[pl.ds(sid * zpw, zpw)],
                        out_hbm.at[pl.ds(cid * S * S + sid * zpw, zpw)])

    return sc_scatter


def _sum_counts(c_ref):
    counts = c_ref[0]
    for c in range(1, c_ref.shape[0]):
        counts = counts + c_ref[c]
    return counts


def _stage1_body(c_ref, x_ref, w1s_ref, w1n_ref, b1_ref, out_ref,
                 mask_scr, m1_scr):
    Ti = out_ref.shape[0]
    S = mask_scr.shape[0]

    @pl.when(pl.program_id(0) == 0)
    def _():
        A = (_sum_counts(c_ref) > 0.5).astype(jnp.float32)
        row_ids = jax.lax.broadcasted_iota(jnp.int32, (S, 1), 0)
        col_ids = jax.lax.broadcasted_iota(jnp.int32, (1, S), 1)
        eye = (row_ids == col_ids).astype(jnp.float32)
        mask = jnp.minimum(A + eye, 1.0)
        mask_scr[...] = mask.astype(jnp.bfloat16)
        m1_scr[...] = jnp.dot(
            A.astype(jnp.bfloat16), mask.astype(jnp.bfloat16),
            preferred_element_type=jnp.float32).astype(jnp.bfloat16)

    x = x_ref[...].astype(jnp.bfloat16)
    w1s = w1s_ref[...].astype(jnp.bfloat16)
    w1n = w1n_ref[...].astype(jnp.bfloat16)
    b1 = b1_ref[...]
    base = pl.program_id(0) * Ti
    mrows = mask_scr[pl.ds(base, Ti), :]
    nrows = m1_scr[pl.ds(base, Ti), :]
    t1 = (mrows[:, :, None] * w1s[None, :, :]
          + nrows[:, :, None] * w1n[None, :, :])
    for i in range(Ti):
        h = jnp.dot(x, t1[i], preferred_element_type=jnp.float32) + b1
        out_ref[i] = jax.nn.relu(h).astype(out_ref.dtype)


def _stage2_body(h1_ref, c_ref, w2s_ref, w2n_ref, b2_ref, wr_ref,
                 bout_ref, out_ref):
    S, Bt, H = h1_ref.shape
    h3 = h1_ref[...]
    h2 = h3.reshape(S, Bt * H)
    A = (_sum_counts(c_ref) > 0.5).astype(jnp.bfloat16)
    agg = jnp.dot(A, h2,
                  preferred_element_type=jnp.float32)
    hr = h3.reshape(S * Bt, H)
    ar = agg.astype(jnp.bfloat16).reshape(S * Bt, H)
    z = jax.nn.relu(
        jnp.dot(hr, w2s_ref[...].astype(jnp.bfloat16),
                preferred_element_type=jnp.float32)
        + jnp.dot(ar, w2n_ref[...].astype(jnp.bfloat16),
                  preferred_element_type=jnp.float32)
        + b2_ref[...])
    z3 = z.reshape(S, Bt, H)
    y = jnp.sum(z3 * wr_ref[...][:, None, :], axis=(0, 2))
    out_ref[...] = y.reshape(Bt, 1) + bout_ref[0, 0]


def kernel(state, action, edge_index, W1_self, W1_neigh, b1,
           W2_self, W2_neigh, b2, W_out, b_out):
    B = state.shape[0]
    S, H = W1_self.shape
    x = jnp.concatenate([state, action], axis=1)
    E = edge_index.shape[1]

    counts_flat = _make_sc_scatter(S, E)(
        edge_index[0], edge_index[1],
        jnp.ones((E,), jnp.float32), jnp.zeros((S * S,), jnp.float32))
    counts = counts_flat.reshape(-1, S, S)
    NCc = counts.shape[0]

    Ti = 16
    full = lambda shape: pl.BlockSpec(shape, lambda i: (0,) * len(shape))
    h1 = pl.pallas_call(
        _stage1_body,
        grid=(S // Ti,),
        in_specs=[
            full((NCc, S, S)),
            full((B, S)),
            full((S, H)), full((S, H)), full((1, H)),
        ],
        out_specs=pl.BlockSpec((Ti, B, H), lambda i: (i, 0, 0)),
        out_shape=jax.ShapeDtypeStruct((S, B, H), jnp.bfloat16),
        scratch_shapes=[pltpu.VMEM((S, S), jnp.bfloat16),
                        pltpu.VMEM((S, S), jnp.bfloat16)],
    )(counts, x, W1_self, W1_neigh, b1.reshape(1, H))

    Bt = 32
    y = pl.pallas_call(
        _stage2_body,
        grid=(B // Bt,),
        in_specs=[
            pl.BlockSpec((S, Bt, H), lambda j: (0, j, 0)),
            full((NCc, S, S)),
            full((H, H)), full((H, H)), full((1, H)),
            full((S, H)), full((1, 1)),
        ],
        out_specs=pl.BlockSpec((Bt, 1), lambda j: (j, 0)),
        out_shape=jax.ShapeDtypeStruct((B, 1), jnp.float32),
    )(h1, counts, W2_self, W2_neigh, b2.reshape(1, H), W_out.reshape(S, H),
      b_out.reshape(1, 1))
    return y


# speedup vs baseline: 1.0083x; 1.0083x over previous
import functools

import jax
import jax.numpy as jnp
from jax import lax
from jax.experimental import pallas as pl
from jax.experimental.pallas import tpu as pltpu
from jax.experimental.pallas import tpu_sc as plsc


def _make_sc_scatter(S, E):
    info = plsc.get_sparse_core_info()
    NC, NS = 1, info.num_subcores
    NW = NC * NS
    epw = E // NW
    zpw = (S * S) // NS
    mesh = plsc.VectorSubcoreMesh(core_axis_name="c", subcore_axis_name="s",
                                  num_cores=NC)

    @functools.partial(
        pl.kernel, mesh=mesh,
        out_type=jax.ShapeDtypeStruct((NC * S * S,), jnp.float32),
        scratch_types=[
            pltpu.VMEM((epw,), jnp.int32),
            pltpu.VMEM((epw,), jnp.int32),
            pltpu.VMEM((epw,), jnp.int32),
            pltpu.VMEM((epw,), jnp.float32),
            pltpu.VMEM_SHARED((S * S,), jnp.float32),
            pltpu.SemaphoreType.DMA,
        ],
    )
    def sc_scatter(src_hbm, dst_hbm, ones_hbm, zeros_hbm, out_hbm,
                   dst_v, src_v, fidx_v, ones_v, acc, sem):
        cid = lax.axis_index("c")
        sid = lax.axis_index("s")
        wid = sid * NC + cid
        base = wid * epw
        c0 = pltpu.async_copy(zeros_hbm.at[pl.ds(sid * zpw, zpw)],
                              acc.at[pl.ds(sid * zpw, zpw)], sem)
        c1 = pltpu.async_copy(dst_hbm.at[pl.ds(base, epw)], dst_v, sem)
        c2 = pltpu.async_copy(src_hbm.at[pl.ds(base, epw)], src_v, sem)
        c3 = pltpu.async_copy(ones_hbm.at[pl.ds(base, epw)], ones_v, sem)
        c0.wait()
        c1.wait()
        c2.wait()
        c3.wait()
        for j in range(epw // 16):
            sl = pl.ds(j * 16, 16)
            fidx_v[sl] = dst_v[sl] * S + src_v[sl]
        plsc.subcore_barrier()
        pltpu.sync_copy(ones_v, acc.at[fidx_v], add=True)
        plsc.subcore_barrier()
        pltpu.sync_copy(acc.at[pl.ds(sid * zpw, zpw)],
                        out_hbm.at[pl.ds(cid * S * S + sid * zpw, zpw)])

    return sc_scatter


def _sum_counts(c_ref):
    counts = c_ref[0]
    for c in range(1, c_ref.shape[0]):
        counts = counts + c_ref[c]
    return counts


def _stage1_body(c_ref, x_ref, w1s_ref, w1n_ref, b1_ref, out_ref,
                 mask_scr, m1_scr):
    Ti = out_ref.shape[0]
    S = mask_scr.shape[0]

    @pl.when(pl.program_id(0) == 0)
    def _():
        A = (_sum_counts(c_ref) > 0.5).astype(jnp.float32)
        row_ids = jax.lax.broadcasted_iota(jnp.int32, (S, 1), 0)
        col_ids = jax.lax.broadcasted_iota(jnp.int32, (1, S), 1)
        eye = (row_ids == col_ids).astype(jnp.float32)
        mask = jnp.minimum(A + eye, 1.0)
        mask_scr[...] = mask.astype(jnp.bfloat16)
        m1_scr[...] = jnp.dot(
            A.astype(jnp.bfloat16), mask.astype(jnp.bfloat16),
            preferred_element_type=jnp.float32).astype(jnp.bfloat16)

    x = x_ref[...].astype(jnp.bfloat16)
    w1s = w1s_ref[...].astype(jnp.bfloat16)
    w1n = w1n_ref[...].astype(jnp.bfloat16)
    b1 = b1_ref[...]
    base = pl.program_id(0) * Ti
    mrows = mask_scr[pl.ds(base, Ti), :]
    nrows = m1_scr[pl.ds(base, Ti), :]
    t1 = (mrows[:, :, None] * w1s[None, :, :]
          + nrows[:, :, None] * w1n[None, :, :])
    for i in range(Ti):
        h = jnp.dot(x, t1[i], preferred_element_type=jnp.float32) + b1
        out_ref[i] = jax.nn.relu(h).astype(out_ref.dtype)


def _stage2_body(h1_ref, c_ref, w2s_ref, w2n_ref, b2_ref, wr_ref,
                 bout_ref, out_ref):
    S, Bt, H = h1_ref.shape
    h3 = h1_ref[...]
    h2 = h3.reshape(S, Bt * H)
    A = (_sum_counts(c_ref) > 0.5).astype(jnp.bfloat16)
    agg = jnp.dot(A, h2,
                  preferred_element_type=jnp.float32)
    hr = h3.reshape(S * Bt, H)
    ar = agg.astype(jnp.bfloat16).reshape(S * Bt, H)
    z = jax.nn.relu(
        jnp.dot(hr, w2s_ref[...].astype(jnp.bfloat16),
                preferred_element_type=jnp.float32)
        + jnp.dot(ar, w2n_ref[...].astype(jnp.bfloat16),
                  preferred_element_type=jnp.float32)
        + b2_ref[...])
    z3 = z.reshape(S, Bt, H)
    y = jnp.sum(z3 * wr_ref[...][:, None, :], axis=(0, 2))
    out_ref[...] = y.reshape(Bt, 1) + bout_ref[0, 0]


def kernel(state, action, edge_index, W1_self, W1_neigh, b1,
           W2_self, W2_neigh, b2, W_out, b_out):
    B = state.shape[0]
    S, H = W1_self.shape
    x = jnp.concatenate([state, action], axis=1)
    E = edge_index.shape[1]

    counts_flat = _make_sc_scatter(S, E)(
        edge_index[0], edge_index[1],
        jnp.ones((E,), jnp.float32), jnp.zeros((S * S,), jnp.float32))
    counts = counts_flat.reshape(-1, S, S)
    NCc = counts.shape[0]

    Ti = 16
    full = lambda shape: pl.BlockSpec(shape, lambda i: (0,) * len(shape))
    h1 = pl.pallas_call(
        _stage1_body,
        grid=(S // Ti,),
        in_specs=[
            full((NCc, S, S)),
            full((B, S)),
            full((S, H)), full((S, H)), full((1, H)),
        ],
        out_specs=pl.BlockSpec((Ti, B, H), lambda i: (i, 0, 0)),
        out_shape=jax.ShapeDtypeStruct((S, B, H), jnp.bfloat16),
        scratch_shapes=[pltpu.VMEM((S, S), jnp.bfloat16),
                        pltpu.VMEM((S, S), jnp.bfloat16)],
    )(counts, x, W1_self, W1_neigh, b1.reshape(1, H))

    Bt = 64
    y = pl.pallas_call(
        _stage2_body,
        grid=(B // Bt,),
        in_specs=[
            pl.BlockSpec((S, Bt, H), lambda j: (0, j, 0)),
            full((NCc, S, S)),
            full((H, H)), full((H, H)), full((1, H)),
            full((S, H)), full((1, 1)),
        ],
        out_specs=pl.BlockSpec((Bt, 1), lambda j: (j, 0)),
        out_shape=jax.ShapeDtypeStruct((B, 1), jnp.float32),
    )(h1, counts, W2_self, W2_neigh, b2.reshape(1, H), W_out.reshape(S, H),
      b_out.reshape(1, 1))
    return y
